# initial kernel scaffold (unmeasured)
import jax
import jax.numpy as jnp
from jax import lax
from jax.experimental import pallas as pl
from jax.experimental.pallas import tpu as pltpu

N_DEV = 16
M_PER = 256
N_PER = 128
K = 4096
N_TOT = 2048


def kernel(x, w_mat, scale_x, scale_w):
    m_per, k = x.shape
    n_tot = w_mat.shape[1]
    n_per = n_tot // N_DEV
    scale = (scale_x[0] * scale_w[0]).reshape(1, 1).astype(jnp.float32)

    def body(x_ref, w_ref, scale_ref, out_ref, comm_ref, xb_ref,
             send_sems, recv_sems):
        me = lax.axis_index("i")
        s = scale_ref[0, 0]

        xb_ref[...] = x_ref[...].astype(jnp.bfloat16)

        def compute_block(j):
            wblk = w_ref[:, pl.ds(j * n_per, n_per)].astype(jnp.bfloat16)
            acc = jnp.dot(xb_ref[...], wblk,
                          preferred_element_type=jnp.float32)
            y = acc * s
            return y / (1.0 + jnp.exp(-jnp.clip(y, -60.0, 60.0)))

        rdmas = []
        for hop in range(1, N_DEV):
            dst = (me + hop) % N_DEV
            comm_ref[hop, :, :] = compute_block(dst)
            rdma = pltpu.make_async_remote_copy(
                src_ref=comm_ref.at[hop],
                dst_ref=out_ref.at[pl.ds(me * m_per, m_per), :],
                send_sem=send_sems.at[hop],
                recv_sem=recv_sems.at[hop],
                device_id=(dst,),
                device_id_type=pl.DeviceIdType.MESH,
            )
            rdma.start()
            rdmas.append(rdma)

        out_ref[pl.ds(me * m_per, m_per), :] = compute_block(me)

        for hop in range(1, N_DEV):
            src = (me - hop) % N_DEV
            recv = pltpu.make_async_remote_copy(
                src_ref=comm_ref.at[hop],
                dst_ref=out_ref.at[pl.ds(src * m_per, m_per), :],
                send_sem=send_sems.at[hop],
                recv_sem=recv_sems.at[hop],
                device_id=(src,),
                device_id_type=pl.DeviceIdType.MESH,
            )
            recv.wait_recv()
        for rdma in rdmas:
            rdma.wait_send()

    return pl.pallas_call(
        body,
        out_shape=jax.ShapeDtypeStruct((N_DEV * m_per, n_per), jnp.float32),
        in_specs=[
            pl.BlockSpec(memory_space=pltpu.VMEM),
            pl.BlockSpec(memory_space=pltpu.VMEM),
            pl.BlockSpec(memory_space=pltpu.SMEM),
        ],
        out_specs=pl.BlockSpec(memory_space=pltpu.VMEM),
        scratch_shapes=[
            pltpu.VMEM((N_DEV, m_per, n_per), jnp.float32),
            pltpu.VMEM((m_per, k), jnp.bfloat16),
            pltpu.SemaphoreType.DMA((N_DEV,)),
            pltpu.SemaphoreType.DMA((N_DEV,)),
        ],
        compiler_params=pltpu.CompilerParams(collective_id=0),
    )(x, w_mat, scale)


# baseline (device time: 45787 ns/iter reference)
import jax
import jax.numpy as jnp
from jax import lax
from jax.experimental import pallas as pl
from jax.experimental.pallas import tpu as pltpu

N_DEV = 16
M_PER = 256
N_PER = 128
K = 4096
N_TOT = 2048


def kernel(x, w_mat, scale_x, scale_w):
    m_per, k = x.shape
    n_tot = w_mat.shape[1]
    n_per = n_tot // N_DEV
    scale = (scale_x[0] * scale_w[0]).reshape(1, 1).astype(jnp.float32)

    def body(x_ref, w_ref, scale_ref, out_ref, comm_ref, xb_ref,
             send_sems, recv_sems):
        me = lax.axis_index("i")
        s = scale_ref[0, 0]

        barrier_sem = pltpu.get_barrier_semaphore()
        for hop in range(1, N_DEV):
            pl.semaphore_signal(
                barrier_sem, inc=1,
                device_id=((me + hop) % N_DEV,),
                device_id_type=pl.DeviceIdType.MESH,
            )
        pl.semaphore_wait(barrier_sem, N_DEV - 1)

        xb_ref[...] = x_ref[...].astype(jnp.bfloat16)

        def compute_block(j):
            wblk = w_ref[:, pl.ds(j * n_per, n_per)].astype(jnp.bfloat16)
            acc = jnp.dot(xb_ref[...], wblk,
                          preferred_element_type=jnp.float32)
            y = acc * s
            return y / (1.0 + jnp.exp(-jnp.clip(y, -60.0, 60.0)))

        rdmas = []
        for hop in range(1, N_DEV):
            dst = (me + hop) % N_DEV
            comm_ref[hop, :, :] = compute_block(dst)
            rdma = pltpu.make_async_remote_copy(
                src_ref=comm_ref.at[hop],
                dst_ref=out_ref.at[pl.ds(me * m_per, m_per), :],
                send_sem=send_sems.at[hop],
                recv_sem=recv_sems.at[hop],
                device_id=(dst,),
                device_id_type=pl.DeviceIdType.MESH,
            )
            rdma.start()
            rdmas.append(rdma)

        out_ref[pl.ds(me * m_per, m_per), :] = compute_block(me)

        for hop in range(1, N_DEV):
            src = (me - hop) % N_DEV
            recv = pltpu.make_async_remote_copy(
                src_ref=comm_ref.at[hop],
                dst_ref=out_ref.at[pl.ds(src * m_per, m_per), :],
                send_sem=send_sems.at[hop],
                recv_sem=recv_sems.at[hop],
                device_id=(src,),
                device_id_type=pl.DeviceIdType.MESH,
            )
            recv.wait_recv()
        for rdma in rdmas:
            rdma.wait_send()

    return pl.pallas_call(
        body,
        out_shape=jax.ShapeDtypeStruct((N_DEV * m_per, n_per), jnp.float32),
        in_specs=[
            pl.BlockSpec(memory_space=pltpu.VMEM),
            pl.BlockSpec(memory_space=pltpu.VMEM),
            pl.BlockSpec(memory_space=pltpu.SMEM),
        ],
        out_specs=pl.BlockSpec(memory_space=pltpu.VMEM),
        scratch_shapes=[
            pltpu.VMEM((N_DEV, m_per, n_per), jnp.float32),
            pltpu.VMEM((m_per, k), jnp.bfloat16),
            pltpu.SemaphoreType.DMA((N_DEV,)),
            pltpu.SemaphoreType.DMA((N_DEV,)),
        ],
        compiler_params=pltpu.CompilerParams(
            collective_id=0, vmem_limit_bytes=100 * 1024 * 1024
        ),
    )(x, w_mat, scale)


# device time: 40351 ns/iter; 1.1347x vs baseline; 1.1347x over previous
import jax
import jax.numpy as jnp
from jax import lax
from jax.experimental import pallas as pl
from jax.experimental.pallas import tpu as pltpu

N_DEV = 16


def kernel(x, w_mat, scale_x, scale_w):
    m_per, k = x.shape
    n_tot = w_mat.shape[1]
    n_per = n_tot // N_DEV
    scale = (scale_x[0] * scale_w[0]).reshape(1, 1).astype(jnp.float32)

    def body(x_ref, w_hbm, scale_ref, out_ref, comm_ref, recv_ref, xb_ref,
             wbuf_ref, load_sems, send_sems, recv_sems):
        me = lax.axis_index("i")
        s = scale_ref[0, 0]

        barrier_sem = pltpu.get_barrier_semaphore()
        for hop in range(1, N_DEV):
            pl.semaphore_signal(
                barrier_sem, inc=1,
                device_id=((me + hop) % N_DEV,),
                device_id_type=pl.DeviceIdType.MESH,
            )
        pl.semaphore_wait(barrier_sem, N_DEV - 1)

        hops = list(range(1, N_DEV)) + [0]

        def w_load(hop, slot):
            j = (me + hop) % N_DEV
            return pltpu.make_async_copy(
                w_hbm.at[:, pl.ds(j * n_per, n_per)],
                wbuf_ref.at[slot],
                load_sems.at[slot],
            )

        w_load(hops[0], 0).start()

        xb_ref[...] = x_ref[...].astype(jnp.bfloat16)

        rdmas = []
        for t, hop in enumerate(hops):
            slot = t % 2
            w_load(hop, slot).wait()
            if t + 1 < N_DEV:
                w_load(hops[t + 1], slot ^ 1).start()
            acc = jnp.dot(xb_ref[...], wbuf_ref[slot].astype(jnp.bfloat16),
                          preferred_element_type=jnp.float32)
            y = acc * s
            y = y / (1.0 + jnp.exp(-jnp.clip(y, -60.0, 60.0)))
            if hop == 0:
                out_ref[pl.ds(me * m_per, m_per), :] = y
            else:
                comm_ref[hop, :, :] = y.astype(jnp.bfloat16)
                rdma = pltpu.make_async_remote_copy(
                    src_ref=comm_ref.at[hop],
                    dst_ref=recv_ref.at[hop],
                    send_sem=send_sems.at[hop],
                    recv_sem=recv_sems.at[hop],
                    device_id=((me + hop) % N_DEV,),
                    device_id_type=pl.DeviceIdType.MESH,
                )
                rdma.start()
                rdmas.append(rdma)

        for hop in range(1, N_DEV):
            src = (me - hop) % N_DEV
            recv = pltpu.make_async_remote_copy(
                src_ref=comm_ref.at[hop],
                dst_ref=recv_ref.at[hop],
                send_sem=send_sems.at[hop],
                recv_sem=recv_sems.at[hop],
                device_id=(src,),
                device_id_type=pl.DeviceIdType.MESH,
            )
            recv.wait_recv()
            out_ref[pl.ds(src * m_per, m_per), :] = (
                recv_ref[hop, :, :].astype(jnp.float32))
        for rdma in rdmas:
            rdma.wait_send()

    return pl.pallas_call(
        body,
        out_shape=jax.ShapeDtypeStruct((N_DEV * m_per, n_per), jnp.float32),
        in_specs=[
            pl.BlockSpec(memory_space=pltpu.VMEM),
            pl.BlockSpec(memory_space=pltpu.MemorySpace.HBM),
            pl.BlockSpec(memory_space=pltpu.SMEM),
        ],
        out_specs=pl.BlockSpec(memory_space=pltpu.VMEM),
        scratch_shapes=[
            pltpu.VMEM((N_DEV, m_per, n_per), jnp.bfloat16),
            pltpu.VMEM((N_DEV, m_per, n_per), jnp.bfloat16),
            pltpu.VMEM((m_per, k), jnp.bfloat16),
            pltpu.VMEM((2, k, n_per), jnp.float32),
            pltpu.SemaphoreType.DMA((2,)),
            pltpu.SemaphoreType.DMA((N_DEV,)),
            pltpu.SemaphoreType.DMA((N_DEV,)),
        ],
        compiler_params=pltpu.CompilerParams(
            collective_id=0, vmem_limit_bytes=100 * 1024 * 1024
        ),
    )(x, w_mat, scale)


# device time: 30863 ns/iter; 1.4836x vs baseline; 1.3074x over previous
import os

import jax
import jax.numpy as jnp
from jax import lax
from jax.experimental import pallas as pl
from jax.experimental.pallas import tpu as pltpu

N_DEV = 16
_VARIANT = os.environ.get("SCBAND_VARIANT", "")


def kernel(x, w_mat, scale_x, scale_w):
    m_per, k = x.shape
    n_tot = w_mat.shape[1]
    n_per = n_tot // N_DEV
    scale = (scale_x[0] * scale_w[0]).reshape(1, 1).astype(jnp.float32)

    def body(x_ref, w_hbm, scale_ref, out_ref, comm_ref, recv_ref, xb_ref,
             wbuf_ref, load_sems, send_sems, recv_sems):
        me = lax.axis_index("i")
        s = scale_ref[0, 0]

        if _VARIANT != "nocomm":
            barrier_sem = pltpu.get_barrier_semaphore()
            for hop in range(1, N_DEV):
                pl.semaphore_signal(
                    barrier_sem, inc=1,
                    device_id=((me + hop) % N_DEV,),
                    device_id_type=pl.DeviceIdType.MESH,
                )
            pl.semaphore_wait(barrier_sem, N_DEV - 1)

        hops = list(range(1, N_DEV)) + [0]

        def w_load(hop, slot):
            j = (me + hop) % N_DEV
            return pltpu.make_async_copy(
                w_hbm.at[:, pl.ds(j * n_per, n_per)],
                wbuf_ref.at[slot],
                load_sems.at[slot],
            )

        w_load(hops[0], 0).start()

        xb_ref[...] = x_ref[...].astype(jnp.bfloat16)

        if _VARIANT == "nocompute":
            comm_ref[...] = jnp.zeros_like(comm_ref)
        rdmas = []
        for t, hop in enumerate(hops):
            slot = t % 2
            if _VARIANT != "nocompute":
                w_load(hop, slot).wait()
                if t + 1 < N_DEV:
                    w_load(hops[t + 1], slot ^ 1).start()
                acc = jnp.dot(xb_ref[...],
                              wbuf_ref[slot].astype(jnp.bfloat16),
                              preferred_element_type=jnp.float32)
                y = acc * s
                y = y / (1.0 + jnp.exp(-jnp.clip(y, -60.0, 60.0)))
                if hop == 0:
                    out_ref[pl.ds(me * m_per, m_per), :] = y
                else:
                    comm_ref[hop, :, :] = y.astype(jnp.bfloat16)
            if hop != 0 and _VARIANT != "nocomm":
                rdma = pltpu.make_async_remote_copy(
                    src_ref=comm_ref.at[hop],
                    dst_ref=recv_ref.at[hop],
                    send_sem=send_sems.at[hop],
                    recv_sem=recv_sems.at[hop],
                    device_id=((me + hop) % N_DEV,),
                    device_id_type=pl.DeviceIdType.MESH,
                )
                rdma.start()
                rdmas.append(rdma)

        for hop in range(1, N_DEV):
            src = (me - hop) % N_DEV
            if _VARIANT != "nocomm":
                recv = pltpu.make_async_remote_copy(
                    src_ref=comm_ref.at[hop],
                    dst_ref=recv_ref.at[hop],
                    send_sem=send_sems.at[hop],
                    recv_sem=recv_sems.at[hop],
                    device_id=(src,),
                    device_id_type=pl.DeviceIdType.MESH,
                )
                recv.wait_recv()
            out_ref[pl.ds(src * m_per, m_per), :] = (
                recv_ref[hop, :, :].astype(jnp.float32))
        for rdma in rdmas:
            rdma.wait_send()

    return pl.pallas_call(
        body,
        out_shape=jax.ShapeDtypeStruct((N_DEV * m_per, n_per), jnp.float32),
        in_specs=[
            pl.BlockSpec(memory_space=pltpu.VMEM),
            pl.BlockSpec(memory_space=pltpu.MemorySpace.HBM),
            pl.BlockSpec(memory_space=pltpu.SMEM),
        ],
        out_specs=pl.BlockSpec(memory_space=pltpu.VMEM),
        scratch_shapes=[
            pltpu.VMEM((N_DEV, m_per, n_per), jnp.bfloat16),
            pltpu.VMEM((N_DEV, m_per, n_per), jnp.bfloat16),
            pltpu.VMEM((m_per, k), jnp.bfloat16),
            pltpu.VMEM((2, k, n_per), jnp.float32),
            pltpu.SemaphoreType.DMA((2,)),
            pltpu.SemaphoreType.DMA((N_DEV,)),
            pltpu.SemaphoreType.DMA((N_DEV,)),
        ],
        compiler_params=pltpu.CompilerParams(
            collective_id=None if _VARIANT == "nocomm" else 0,
            vmem_limit_bytes=100 * 1024 * 1024,
        ),
    )(x, w_mat, scale)
